# baseline (speedup 1.0000x reference)
import functools

import jax
import jax.numpy as jnp
from jax import lax
from jax.experimental import pallas as pl
from jax.experimental.pallas import tpu as pltpu
from jax.experimental.pallas import tpu_sc as plsc

TOPK = 5
B = 1024
N = 100000
BR = 8
NSTEP = B // BR


def _gather_tvals(preds, target):
    info = plsc.get_sparse_core_info()
    nc, ns, L = info.num_cores, info.num_subcores, info.num_lanes
    nw = nc * ns
    per_w = B // nw
    groups = per_w // L
    W = 128

    mesh = plsc.VectorSubcoreMesh(core_axis_name="c", subcore_axis_name="s")

    @functools.partial(
        pl.kernel,
        mesh=mesh,
        out_type=jax.ShapeDtypeStruct((B,), jnp.float32),
        scratch_types=[
            pltpu.VMEM((per_w,), jnp.int32),
            pltpu.VMEM((per_w, 8, W), jnp.float32),
            pltpu.VMEM((per_w,), jnp.float32),
            pltpu.SemaphoreType.DMA,
        ],
    )
    def gather_kernel(preds_hbm, tgt_hbm, out_hbm, tgt_v, rows_v, val_v, sem):
        wid = lax.axis_index("s") * nc + lax.axis_index("c")
        base = wid * per_w
        pltpu.sync_copy(tgt_hbm.at[pl.ds(base, per_w)], tgt_v)
        copies = []
        for g in range(groups):
            t16 = tgt_v[pl.ds(g * L, L)]
            for r in range(L):
                t = lax.squeeze(lax.slice(t16, (r,), (r + 1,)), (0,))
                s0 = pl.multiple_of((t >> 7) << 7, W)
                k = g * L + r
                row8 = pl.multiple_of(base + (k & ~7), 8)
                copies.append(
                    pltpu.async_copy(
                        preds_hbm.at[pl.ds(row8, 8), pl.ds(s0, W)],
                        rows_v.at[k], sem,
                    )
                )
        for c in copies:
            c.wait()
        k_iota = lax.iota(jnp.int32, L)
        dnums = lax.GatherDimensionNumbers(
            offset_dims=(), collapsed_slice_dims=(0,), start_index_map=(0,)
        )
        for g in range(groups):
            t16 = tgt_v[pl.ds(g * L, L)]
            lane = lax.bitwise_and(t16, W - 1)
            chunk_of = lax.shift_right_logical(lane, 4)
            lane15 = lax.bitwise_and(lane, 15)
            acc = jnp.zeros((L,), jnp.float32)
            for r in range(L):
                k = g * L + r
                for c in range(W // L):
                    chunk = rows_v[k, k % 8, pl.ds(c * L, L)]
                    sel = lax.gather(
                        chunk, lane15[:, None], dnums, slice_sizes=(1,),
                        mode=lax.GatherScatterMode.PROMISE_IN_BOUNDS,
                    )
                    acc = jnp.where((k_iota == r) & (chunk_of == c), sel, acc)
            val_v[pl.ds(g * L, L)] = acc
        pltpu.sync_copy(val_v, out_hbm.at[pl.ds(base, per_w)])

    return gather_kernel(preds, target)


def _count_body(pred_ref, tval_ref, tgt_ref, out_ref, acc_ref):
    j = pl.program_id(0)

    @pl.when(j == 0)
    def _():
        acc_ref[...] = jnp.zeros_like(acc_ref)

    blk = pred_ref[...]
    tval = tval_ref[...]
    tgt = tgt_ref[...]
    cols = lax.broadcasted_iota(jnp.int32, (BR, N), 1)
    hit = (blk > tval) | ((blk == tval) & (cols < tgt))
    rank = jnp.sum(hit.astype(jnp.int32), axis=1, keepdims=True)
    correct = (rank < TOPK).astype(jnp.float32)
    acc_ref[...] += jnp.sum(correct, axis=(0, 1), keepdims=True)

    @pl.when(j == NSTEP - 1)
    def _():
        out_ref[...] = acc_ref[...] * (1.0 / B)


def kernel(preds, target):
    tvals = _gather_tvals(preds, target)
    out = pl.pallas_call(
        _count_body,
        grid=(NSTEP,),
        in_specs=[
            pl.BlockSpec((BR, N), lambda j: (j, 0)),
            pl.BlockSpec((BR, 1), lambda j: (j, 0)),
            pl.BlockSpec((BR, 1), lambda j: (j, 0)),
        ],
        out_specs=pl.BlockSpec((1, 1), lambda j: (0, 0)),
        out_shape=jax.ShapeDtypeStruct((1, 1), jnp.float32),
        scratch_shapes=[pltpu.VMEM((1, 1), jnp.float32)],
    )(preds, tvals.reshape(B, 1), target.reshape(B, 1).astype(jnp.int32))
    return out[0, 0]


# speedup vs baseline: 1.0001x; 1.0001x over previous
import functools

import jax
import jax.numpy as jnp
from jax import lax
from jax.experimental import pallas as pl
from jax.experimental.pallas import tpu as pltpu
from jax.experimental.pallas import tpu_sc as plsc

TOPK = 5
B = 1024
N = 100000
BR = 8
NSTEP = B // BR


def _gather_tvals(preds, target):
    info = plsc.get_sparse_core_info()
    nc, ns, L = info.num_cores, info.num_subcores, info.num_lanes
    nw = nc * ns
    per_w = B // nw
    groups = per_w // L
    W = 128

    mesh = plsc.VectorSubcoreMesh(core_axis_name="c", subcore_axis_name="s")

    @functools.partial(
        pl.kernel,
        mesh=mesh,
        out_type=jax.ShapeDtypeStruct((B,), jnp.float32),
        scratch_types=[
            pltpu.VMEM((per_w,), jnp.int32),
            pltpu.VMEM((per_w, 8, W), jnp.float32),
            pltpu.VMEM((per_w,), jnp.float32),
            pltpu.SemaphoreType.DMA,
        ],
    )
    def gather_kernel(preds_hbm, tgt_hbm, out_hbm, tgt_v, rows_v, val_v, sem):
        wid = lax.axis_index("s") * nc + lax.axis_index("c")
        base = wid * per_w
        pltpu.sync_copy(tgt_hbm.at[pl.ds(base, per_w)], tgt_v)
        copies = []
        for g in range(groups):
            t16 = tgt_v[pl.ds(g * L, L)]
            for r in range(L):
                t = lax.squeeze(lax.slice(t16, (r,), (r + 1,)), (0,))
                s0 = pl.multiple_of((t >> 7) << 7, W)
                k = g * L + r
                row8 = pl.multiple_of(base + (k & ~7), 8)
                copies.append(
                    pltpu.async_copy(
                        preds_hbm.at[pl.ds(row8, 8), pl.ds(s0, W)],
                        rows_v.at[k], sem,
                    )
                )
        for c in copies:
            c.wait()
        k_iota = lax.iota(jnp.int32, L)
        dnums = lax.GatherDimensionNumbers(
            offset_dims=(), collapsed_slice_dims=(0,), start_index_map=(0,)
        )
        for g in range(groups):
            t16 = tgt_v[pl.ds(g * L, L)]
            lane = lax.bitwise_and(t16, W - 1)
            chunk_of = lax.shift_right_logical(lane, 4)
            lane15 = lax.bitwise_and(lane, 15)
            acc = jnp.zeros((L,), jnp.float32)
            for r in range(L):
                k = g * L + r
                for c in range(W // L):
                    chunk = rows_v[k, k % 8, pl.ds(c * L, L)]
                    sel = lax.gather(
                        chunk, lane15[:, None], dnums, slice_sizes=(1,),
                        mode=lax.GatherScatterMode.PROMISE_IN_BOUNDS,
                    )
                    acc = jnp.where((k_iota == r) & (chunk_of == c), sel, acc)
            val_v[pl.ds(g * L, L)] = acc
        pltpu.sync_copy(val_v, out_hbm.at[pl.ds(base, per_w)])

    return gather_kernel(preds, target)


NL = 50048


def _count_body(predl_ref, predr_ref, tval_ref, tgt_ref, out_ref, acc_ref):
    j = pl.program_id(0)

    @pl.when(j == 0)
    def _():
        acc_ref[...] = jnp.zeros_like(acc_ref)

    tval = tval_ref[...]
    tgt = tgt_ref[...]
    cols = lax.broadcasted_iota(jnp.int32, (BR, NL), 1)
    blkl = predl_ref[...]
    hitl = (blkl > tval) | ((blkl == tval) & (cols < tgt))
    blkr = predr_ref[...]
    colsr = cols + NL
    hitr = ((blkr > tval) & (colsr < N)) | ((blkr == tval) & (colsr < tgt))
    rank = jnp.sum(hitl.astype(jnp.int32), axis=1, keepdims=True)
    rank += jnp.sum(hitr.astype(jnp.int32), axis=1, keepdims=True)
    correct = (rank < TOPK).astype(jnp.float32)
    acc_ref[...] += jnp.sum(correct, axis=(0, 1), keepdims=True)

    @pl.when(j == NSTEP - 1)
    def _():
        out_ref[...] = acc_ref[...] * (1.0 / B)


def kernel(preds, target):
    tvals = _gather_tvals(preds, target)
    out = pl.pallas_call(
        _count_body,
        grid=(NSTEP,),
        in_specs=[
            pl.BlockSpec((BR, NL), lambda j: (j, 0)),
            pl.BlockSpec((BR, NL), lambda j: (j, 1)),
            pl.BlockSpec((BR, 1), lambda j: (j, 0)),
            pl.BlockSpec((BR, 1), lambda j: (j, 0)),
        ],
        out_specs=pl.BlockSpec((1, 1), lambda j: (0, 0)),
        out_shape=jax.ShapeDtypeStruct((1, 1), jnp.float32),
        scratch_shapes=[pltpu.VMEM((1, 1), jnp.float32)],
    )(preds, preds, tvals.reshape(B, 1), target.reshape(B, 1).astype(jnp.int32))
    return out[0, 0]


# speedup vs baseline: 1.0854x; 1.0853x over previous
import functools

import jax
import jax.numpy as jnp
from jax import lax
from jax.experimental import pallas as pl
from jax.experimental.pallas import tpu as pltpu
from jax.experimental.pallas import tpu_sc as plsc

TOPK = 5
B = 1024
N = 100000
BN = 2048
NBLK = (N + BN - 1) // BN


def _gather_tvals(preds, target):
    info = plsc.get_sparse_core_info()
    nc, ns, L = info.num_cores, info.num_subcores, info.num_lanes
    nw = nc * ns
    per_w = B // nw
    groups = per_w // L
    W = 128

    mesh = plsc.VectorSubcoreMesh(core_axis_name="c", subcore_axis_name="s")

    @functools.partial(
        pl.kernel,
        mesh=mesh,
        out_type=jax.ShapeDtypeStruct((B,), jnp.float32),
        scratch_types=[
            pltpu.VMEM((per_w,), jnp.int32),
            pltpu.VMEM((per_w, 8, W), jnp.float32),
            pltpu.VMEM((per_w,), jnp.float32),
            pltpu.SemaphoreType.DMA,
        ],
    )
    def gather_kernel(preds_hbm, tgt_hbm, out_hbm, tgt_v, rows_v, val_v, sem):
        wid = lax.axis_index("s") * nc + lax.axis_index("c")
        base = wid * per_w
        pltpu.sync_copy(tgt_hbm.at[pl.ds(base, per_w)], tgt_v)
        copies = []
        for g in range(groups):
            t16 = tgt_v[pl.ds(g * L, L)]
            for r in range(L):
                t = lax.squeeze(lax.slice(t16, (r,), (r + 1,)), (0,))
                s0 = pl.multiple_of((t >> 7) << 7, W)
                k = g * L + r
                row8 = pl.multiple_of(base + (k & ~7), 8)
                copies.append(
                    pltpu.async_copy(
                        preds_hbm.at[pl.ds(row8, 8), pl.ds(s0, W)],
                        rows_v.at[k], sem,
                    )
                )
        for c in copies:
            c.wait()
        k_iota = lax.iota(jnp.int32, L)
        dnums = lax.GatherDimensionNumbers(
            offset_dims=(), collapsed_slice_dims=(0,), start_index_map=(0,)
        )
        for g in range(groups):
            t16 = tgt_v[pl.ds(g * L, L)]
            lane = lax.bitwise_and(t16, W - 1)
            chunk_of = lax.shift_right_logical(lane, 4)
            lane15 = lax.bitwise_and(lane, 15)
            acc = jnp.zeros((L,), jnp.float32)
            for r in range(L):
                k = g * L + r
                for c in range(W // L):
                    chunk = rows_v[k, k % 8, pl.ds(c * L, L)]
                    sel = lax.gather(
                        chunk, lane15[:, None], dnums, slice_sizes=(1,),
                        mode=lax.GatherScatterMode.PROMISE_IN_BOUNDS,
                    )
                    acc = jnp.where((k_iota == r) & (chunk_of == c), sel, acc)
            val_v[pl.ds(g * L, L)] = acc
        pltpu.sync_copy(val_v, out_hbm.at[pl.ds(base, per_w)])

    return gather_kernel(preds, target)


def _count_body(pred_ref, tval_ref, tgt_ref, out_ref, acc_ref):
    j = pl.program_id(0)

    @pl.when(j == 0)
    def _():
        acc_ref[...] = jnp.zeros_like(acc_ref)

    @pl.when(j < NBLK - 1)
    def _():
        blk = pred_ref[...]
        tval = tval_ref[...]
        tgt = tgt_ref[...]
        cols = j * BN + lax.broadcasted_iota(jnp.int32, (B, BN), 1)
        hit = (blk > tval) | ((blk == tval) & (cols < tgt))
        acc_ref[...] += jnp.sum(hit.astype(jnp.int32), axis=1, keepdims=True)

    @pl.when(j == NBLK - 1)
    def _():
        blk = pred_ref[...]
        tval = tval_ref[...]
        tgt = tgt_ref[...]
        cols = j * BN + lax.broadcasted_iota(jnp.int32, (B, BN), 1)
        hit = ((blk > tval) & (cols < N)) | ((blk == tval) & (cols < tgt))
        acc_ref[...] += jnp.sum(hit.astype(jnp.int32), axis=1, keepdims=True)
        correct = (acc_ref[...] < TOPK).astype(jnp.float32)
        out_ref[...] = jnp.sum(correct, axis=(0, 1), keepdims=True) * (1.0 / B)


def kernel(preds, target):
    tvals = _gather_tvals(preds, target)
    out = pl.pallas_call(
        _count_body,
        grid=(NBLK,),
        in_specs=[
            pl.BlockSpec((B, BN), lambda j: (0, j)),
            pl.BlockSpec((B, 1), lambda j: (0, 0)),
            pl.BlockSpec((B, 1), lambda j: (0, 0)),
        ],
        out_specs=pl.BlockSpec((1, 1), lambda j: (0, 0)),
        out_shape=jax.ShapeDtypeStruct((1, 1), jnp.float32),
        scratch_shapes=[pltpu.VMEM((B, 1), jnp.int32)],
    )(preds, tvals.reshape(B, 1), target.reshape(B, 1).astype(jnp.int32))
    return out[0, 0]


# speedup vs baseline: 1.0886x; 1.0029x over previous
import functools

import jax
import jax.numpy as jnp
from jax import lax
from jax.experimental import pallas as pl
from jax.experimental.pallas import tpu as pltpu
from jax.experimental.pallas import tpu_sc as plsc

TOPK = 5
B = 1024
N = 100000
BN = 2048
NBLK = (N + BN - 1) // BN


def _gather_tvals(preds, target):
    info = plsc.get_sparse_core_info()
    nc, ns, L = info.num_cores, info.num_subcores, info.num_lanes
    nw = nc * ns
    per_w = B // nw
    groups = per_w // L
    W = 128

    mesh = plsc.VectorSubcoreMesh(core_axis_name="c", subcore_axis_name="s")

    @functools.partial(
        pl.kernel,
        mesh=mesh,
        out_type=jax.ShapeDtypeStruct((B,), jnp.float32),
        scratch_types=[
            pltpu.VMEM((per_w,), jnp.int32),
            pltpu.VMEM((per_w, 8, W), jnp.float32),
            pltpu.VMEM((per_w,), jnp.float32),
            pltpu.SemaphoreType.DMA,
        ],
    )
    def gather_kernel(preds_hbm, tgt_hbm, out_hbm, tgt_v, rows_v, val_v, sem):
        wid = lax.axis_index("s") * nc + lax.axis_index("c")
        base = wid * per_w
        pltpu.sync_copy(tgt_hbm.at[pl.ds(base, per_w)], tgt_v)
        copies = []
        for g in range(groups):
            t16 = tgt_v[pl.ds(g * L, L)]
            for r in range(L):
                t = lax.squeeze(lax.slice(t16, (r,), (r + 1,)), (0,))
                s0 = pl.multiple_of((t >> 7) << 7, W)
                k = g * L + r
                row8 = pl.multiple_of(base + (k & ~7), 8)
                copies.append(
                    pltpu.async_copy(
                        preds_hbm.at[pl.ds(row8, 8), pl.ds(s0, W)],
                        rows_v.at[k], sem,
                    )
                )
        for c in copies:
            c.wait()
        k_iota = lax.iota(jnp.int32, L)
        dnums = lax.GatherDimensionNumbers(
            offset_dims=(), collapsed_slice_dims=(0,), start_index_map=(0,)
        )
        for g in range(groups):
            t16 = tgt_v[pl.ds(g * L, L)]
            lane = lax.bitwise_and(t16, W - 1)
            chunk_of = lax.shift_right_logical(lane, 4)
            lane15 = lax.bitwise_and(lane, 15)
            acc = jnp.zeros((L,), jnp.float32)
            for r in range(L):
                k = g * L + r
                for c in range(W // L):
                    chunk = rows_v[k, k % 8, pl.ds(c * L, L)]
                    sel = lax.gather(
                        chunk, lane15[:, None], dnums, slice_sizes=(1,),
                        mode=lax.GatherScatterMode.PROMISE_IN_BOUNDS,
                    )
                    acc = jnp.where((k_iota == r) & (chunk_of == c), sel, acc)
            val_v[pl.ds(g * L, L)] = acc
        pltpu.sync_copy(val_v, out_hbm.at[pl.ds(base, per_w)])

    return gather_kernel(preds, target)


def _count_body(pred_ref, tval_ref, tgt_ref, out_ref, acc_ref):
    j = pl.program_id(0)

    @pl.when(j == 0)
    def _():
        acc_ref[...] = jnp.zeros_like(acc_ref)

    @pl.when(j < NBLK - 1)
    def _():
        blk = pred_ref[...]
        tval = tval_ref[...]
        tgt_adj = tgt_ref[...] - j * BN
        cols = lax.broadcasted_iota(jnp.int32, (B, BN), 1)
        hit = (blk > tval) | ((blk == tval) & (cols < tgt_adj))
        acc_ref[...] += jnp.sum(hit.astype(jnp.int32), axis=1, keepdims=True)

    @pl.when(j == NBLK - 1)
    def _():
        blk = pred_ref[...]
        tval = tval_ref[...]
        tgt_adj = tgt_ref[...] - j * BN
        cols = lax.broadcasted_iota(jnp.int32, (B, BN), 1)
        hit = ((blk > tval) & (cols < N - (NBLK - 1) * BN)) | (
            (blk == tval) & (cols < tgt_adj)
        )
        acc_ref[...] += jnp.sum(hit.astype(jnp.int32), axis=1, keepdims=True)
        correct = (acc_ref[...] < TOPK).astype(jnp.float32)
        out_ref[...] = jnp.sum(correct, axis=(0, 1), keepdims=True) * (1.0 / B)


def kernel(preds, target):
    tvals = _gather_tvals(preds, target)
    out = pl.pallas_call(
        _count_body,
        grid=(NBLK,),
        in_specs=[
            pl.BlockSpec((B, BN), lambda j: (0, j)),
            pl.BlockSpec((B, 1), lambda j: (0, 0)),
            pl.BlockSpec((B, 1), lambda j: (0, 0)),
        ],
        out_specs=pl.BlockSpec((1, 1), lambda j: (0, 0)),
        out_shape=jax.ShapeDtypeStruct((1, 1), jnp.float32),
        scratch_shapes=[pltpu.VMEM((B, 1), jnp.int32)],
    )(preds, tvals.reshape(B, 1), target.reshape(B, 1).astype(jnp.int32))
    return out[0, 0]


# speedup vs baseline: 1.1193x; 1.0282x over previous
import functools

import jax
import jax.numpy as jnp
from jax import lax
from jax.experimental import pallas as pl
from jax.experimental.pallas import tpu as pltpu
from jax.experimental.pallas import tpu_sc as plsc

TOPK = 5
B = 1024
N = 100000
BN = 4096
NBLK = (N + BN - 1) // BN


def _gather_tvals(preds, target):
    info = plsc.get_sparse_core_info()
    nc, ns, L = info.num_cores, info.num_subcores, info.num_lanes
    nw = nc * ns
    per_w = B // nw
    groups = per_w // L
    W = 128

    mesh = plsc.VectorSubcoreMesh(core_axis_name="c", subcore_axis_name="s")

    @functools.partial(
        pl.kernel,
        mesh=mesh,
        out_type=jax.ShapeDtypeStruct((B,), jnp.float32),
        scratch_types=[
            pltpu.VMEM((per_w,), jnp.int32),
            pltpu.VMEM((per_w, 8, W), jnp.float32),
            pltpu.VMEM((per_w,), jnp.float32),
            pltpu.SemaphoreType.DMA,
        ],
    )
    def gather_kernel(preds_hbm, tgt_hbm, out_hbm, tgt_v, rows_v, val_v, sem):
        wid = lax.axis_index("s") * nc + lax.axis_index("c")
        base = wid * per_w
        pltpu.sync_copy(tgt_hbm.at[pl.ds(base, per_w)], tgt_v)
        copies = []
        for g in range(groups):
            t16 = tgt_v[pl.ds(g * L, L)]
            for r in range(L):
                t = lax.squeeze(lax.slice(t16, (r,), (r + 1,)), (0,))
                s0 = pl.multiple_of((t >> 7) << 7, W)
                k = g * L + r
                row8 = pl.multiple_of(base + (k & ~7), 8)
                copies.append(
                    pltpu.async_copy(
                        preds_hbm.at[pl.ds(row8, 8), pl.ds(s0, W)],
                        rows_v.at[k], sem,
                    )
                )
        for c in copies:
            c.wait()
        k_iota = lax.iota(jnp.int32, L)
        dnums = lax.GatherDimensionNumbers(
            offset_dims=(), collapsed_slice_dims=(0,), start_index_map=(0,)
        )
        for g in range(groups):
            t16 = tgt_v[pl.ds(g * L, L)]
            lane = lax.bitwise_and(t16, W - 1)
            chunk_of = lax.shift_right_logical(lane, 4)
            lane15 = lax.bitwise_and(lane, 15)
            acc = jnp.zeros((L,), jnp.float32)
            for r in range(L):
                k = g * L + r
                for c in range(W // L):
                    chunk = rows_v[k, k % 8, pl.ds(c * L, L)]
                    sel = lax.gather(
                        chunk, lane15[:, None], dnums, slice_sizes=(1,),
                        mode=lax.GatherScatterMode.PROMISE_IN_BOUNDS,
                    )
                    acc = jnp.where((k_iota == r) & (chunk_of == c), sel, acc)
            val_v[pl.ds(g * L, L)] = acc
        pltpu.sync_copy(val_v, out_hbm.at[pl.ds(base, per_w)])

    return gather_kernel(preds, target)


def _count_body(pred_ref, tval_ref, tgt_ref, out_ref, acc_ref):
    j = pl.program_id(0)

    @pl.when(j == 0)
    def _():
        acc_ref[...] = jnp.zeros_like(acc_ref)

    @pl.when(j < NBLK - 1)
    def _():
        blk = pred_ref[...]
        tval = tval_ref[...]
        tgt_adj = tgt_ref[...] - j * BN
        cols = lax.broadcasted_iota(jnp.int32, (B, BN), 1)
        hit = (blk > tval) | ((blk == tval) & (cols < tgt_adj))
        acc_ref[...] += jnp.sum(hit.astype(jnp.int32), axis=1, keepdims=True)

    @pl.when(j == NBLK - 1)
    def _():
        blk = pred_ref[...]
        tval = tval_ref[...]
        tgt_adj = tgt_ref[...] - j * BN
        cols = lax.broadcasted_iota(jnp.int32, (B, BN), 1)
        hit = ((blk > tval) & (cols < N - (NBLK - 1) * BN)) | (
            (blk == tval) & (cols < tgt_adj)
        )
        acc_ref[...] += jnp.sum(hit.astype(jnp.int32), axis=1, keepdims=True)
        correct = (acc_ref[...] < TOPK).astype(jnp.float32)
        out_ref[...] = jnp.sum(correct, axis=(0, 1), keepdims=True) * (1.0 / B)


def kernel(preds, target):
    tvals = _gather_tvals(preds, target)
    out = pl.pallas_call(
        _count_body,
        grid=(NBLK,),
        in_specs=[
            pl.BlockSpec((B, BN), lambda j: (0, j)),
            pl.BlockSpec((B, 1), lambda j: (0, 0)),
            pl.BlockSpec((B, 1), lambda j: (0, 0)),
        ],
        out_specs=pl.BlockSpec((1, 1), lambda j: (0, 0)),
        out_shape=jax.ShapeDtypeStruct((1, 1), jnp.float32),
        scratch_shapes=[pltpu.VMEM((B, 1), jnp.int32)],
    )(preds, tvals.reshape(B, 1), target.reshape(B, 1).astype(jnp.int32))
    return out[0, 0]


# speedup vs baseline: 1.1246x; 1.0048x over previous
import functools

import jax
import jax.numpy as jnp
from jax import lax
from jax.experimental import pallas as pl
from jax.experimental.pallas import tpu as pltpu
from jax.experimental.pallas import tpu_sc as plsc

TOPK = 5
B = 1024
N = 100000
BN = 6144
NBLK = (N + BN - 1) // BN


def _gather_tvals(preds, target):
    info = plsc.get_sparse_core_info()
    nc, ns, L = info.num_cores, info.num_subcores, info.num_lanes
    nw = nc * ns
    per_w = B // nw
    groups = per_w // L
    W = 128

    mesh = plsc.VectorSubcoreMesh(core_axis_name="c", subcore_axis_name="s")

    @functools.partial(
        pl.kernel,
        mesh=mesh,
        out_type=jax.ShapeDtypeStruct((B,), jnp.float32),
        scratch_types=[
            pltpu.VMEM((per_w,), jnp.int32),
            pltpu.VMEM((per_w, 8, W), jnp.float32),
            pltpu.VMEM((per_w,), jnp.float32),
            pltpu.SemaphoreType.DMA,
        ],
    )
    def gather_kernel(preds_hbm, tgt_hbm, out_hbm, tgt_v, rows_v, val_v, sem):
        wid = lax.axis_index("s") * nc + lax.axis_index("c")
        base = wid * per_w
        pltpu.sync_copy(tgt_hbm.at[pl.ds(base, per_w)], tgt_v)
        copies = []
        for g in range(groups):
            t16 = tgt_v[pl.ds(g * L, L)]
            for r in range(L):
                t = lax.squeeze(lax.slice(t16, (r,), (r + 1,)), (0,))
                s0 = pl.multiple_of((t >> 7) << 7, W)
                k = g * L + r
                row8 = pl.multiple_of(base + (k & ~7), 8)
                copies.append(
                    pltpu.async_copy(
                        preds_hbm.at[pl.ds(row8, 8), pl.ds(s0, W)],
                        rows_v.at[k], sem,
                    )
                )
        for c in copies:
            c.wait()
        k_iota = lax.iota(jnp.int32, L)
        dnums = lax.GatherDimensionNumbers(
            offset_dims=(), collapsed_slice_dims=(0,), start_index_map=(0,)
        )
        for g in range(groups):
            t16 = tgt_v[pl.ds(g * L, L)]
            lane = lax.bitwise_and(t16, W - 1)
            chunk_of = lax.shift_right_logical(lane, 4)
            lane15 = lax.bitwise_and(lane, 15)
            acc = jnp.zeros((L,), jnp.float32)
            for r in range(L):
                k = g * L + r
                for c in range(W // L):
                    chunk = rows_v[k, k % 8, pl.ds(c * L, L)]
                    sel = lax.gather(
                        chunk, lane15[:, None], dnums, slice_sizes=(1,),
                        mode=lax.GatherScatterMode.PROMISE_IN_BOUNDS,
                    )
                    acc = jnp.where((k_iota == r) & (chunk_of == c), sel, acc)
            val_v[pl.ds(g * L, L)] = acc
        pltpu.sync_copy(val_v, out_hbm.at[pl.ds(base, per_w)])

    return gather_kernel(preds, target)


def _count_body(pred_ref, tval_ref, tgt_ref, out_ref, acc_ref):
    j = pl.program_id(0)

    @pl.when(j == 0)
    def _():
        acc_ref[...] = jnp.zeros_like(acc_ref)

    @pl.when(j < NBLK - 1)
    def _():
        blk = pred_ref[...]
        tval = tval_ref[...]
        tgt_adj = tgt_ref[...] - j * BN
        cols = lax.broadcasted_iota(jnp.int32, (B, BN), 1)
        hit = (blk > tval) | ((blk == tval) & (cols < tgt_adj))
        acc_ref[...] += jnp.sum(hit.astype(jnp.int32), axis=1, keepdims=True)

    @pl.when(j == NBLK - 1)
    def _():
        blk = pred_ref[...]
        tval = tval_ref[...]
        tgt_adj = tgt_ref[...] - j * BN
        cols = lax.broadcasted_iota(jnp.int32, (B, BN), 1)
        hit = ((blk > tval) & (cols < N - (NBLK - 1) * BN)) | (
            (blk == tval) & (cols < tgt_adj)
        )
        acc_ref[...] += jnp.sum(hit.astype(jnp.int32), axis=1, keepdims=True)
        correct = (acc_ref[...] < TOPK).astype(jnp.float32)
        out_ref[...] = jnp.sum(correct, axis=(0, 1), keepdims=True) * (1.0 / B)


def kernel(preds, target):
    tvals = _gather_tvals(preds, target)
    out = pl.pallas_call(
        _count_body,
        grid=(NBLK,),
        in_specs=[
            pl.BlockSpec((B, BN), lambda j: (0, j)),
            pl.BlockSpec((B, 1), lambda j: (0, 0)),
            pl.BlockSpec((B, 1), lambda j: (0, 0)),
        ],
        out_specs=pl.BlockSpec((1, 1), lambda j: (0, 0)),
        out_shape=jax.ShapeDtypeStruct((1, 1), jnp.float32),
        scratch_shapes=[pltpu.VMEM((B, 1), jnp.int32)],
    )(preds, tvals.reshape(B, 1), target.reshape(B, 1).astype(jnp.int32))
    return out[0, 0]


# speedup vs baseline: 1.1327x; 1.0071x over previous
import functools

import jax
import jax.numpy as jnp
from jax import lax
from jax.experimental import pallas as pl
from jax.experimental.pallas import tpu as pltpu
from jax.experimental.pallas import tpu_sc as plsc

TOPK = 5
B = 1024
N = 100000
BN = 7168
NBLK = (N + BN - 1) // BN


def _gather_tvals(preds, target):
    info = plsc.get_sparse_core_info()
    nc, ns, L = info.num_cores, info.num_subcores, info.num_lanes
    nw = nc * ns
    per_w = B // nw
    groups = per_w // L
    W = 128

    mesh = plsc.VectorSubcoreMesh(core_axis_name="c", subcore_axis_name="s")

    @functools.partial(
        pl.kernel,
        mesh=mesh,
        out_type=jax.ShapeDtypeStruct((B,), jnp.float32),
        scratch_types=[
            pltpu.VMEM((per_w,), jnp.int32),
            pltpu.VMEM((per_w, 8, W), jnp.float32),
            pltpu.VMEM((per_w,), jnp.float32),
            pltpu.SemaphoreType.DMA,
        ],
    )
    def gather_kernel(preds_hbm, tgt_hbm, out_hbm, tgt_v, rows_v, val_v, sem):
        wid = lax.axis_index("s") * nc + lax.axis_index("c")
        base = wid * per_w
        pltpu.sync_copy(tgt_hbm.at[pl.ds(base, per_w)], tgt_v)
        copies = []
        for g in range(groups):
            t16 = tgt_v[pl.ds(g * L, L)]
            for r in range(L):
                t = lax.squeeze(lax.slice(t16, (r,), (r + 1,)), (0,))
                s0 = pl.multiple_of((t >> 7) << 7, W)
                k = g * L + r
                row8 = pl.multiple_of(base + (k & ~7), 8)
                copies.append(
                    pltpu.async_copy(
                        preds_hbm.at[pl.ds(row8, 8), pl.ds(s0, W)],
                        rows_v.at[k], sem,
                    )
                )
        for c in copies:
            c.wait()
        k_iota = lax.iota(jnp.int32, L)
        dnums = lax.GatherDimensionNumbers(
            offset_dims=(), collapsed_slice_dims=(0,), start_index_map=(0,)
        )
        for g in range(groups):
            t16 = tgt_v[pl.ds(g * L, L)]
            lane = lax.bitwise_and(t16, W - 1)
            chunk_of = lax.shift_right_logical(lane, 4)
            lane15 = lax.bitwise_and(lane, 15)
            acc = jnp.zeros((L,), jnp.float32)
            for r in range(L):
                k = g * L + r
                for c in range(W // L):
                    chunk = rows_v[k, k % 8, pl.ds(c * L, L)]
                    sel = lax.gather(
                        chunk, lane15[:, None], dnums, slice_sizes=(1,),
                        mode=lax.GatherScatterMode.PROMISE_IN_BOUNDS,
                    )
                    acc = jnp.where((k_iota == r) & (chunk_of == c), sel, acc)
            val_v[pl.ds(g * L, L)] = acc
        pltpu.sync_copy(val_v, out_hbm.at[pl.ds(base, per_w)])

    return gather_kernel(preds, target)


def _count_body(pred_ref, tval_ref, tgt_ref, out_ref, acc_ref):
    j = pl.program_id(0)

    @pl.when(j == 0)
    def _():
        acc_ref[...] = jnp.zeros_like(acc_ref)

    @pl.when(j < NBLK - 1)
    def _():
        blk = pred_ref[...]
        tval = tval_ref[...]
        tgt_adj = tgt_ref[...] - j * BN
        cols = lax.broadcasted_iota(jnp.int32, (B, BN), 1)
        hit = (blk > tval) | ((blk == tval) & (cols < tgt_adj))
        acc_ref[...] += jnp.sum(hit.astype(jnp.int32), axis=1, keepdims=True)

    @pl.when(j == NBLK - 1)
    def _():
        blk = pred_ref[...]
        tval = tval_ref[...]
        tgt_adj = tgt_ref[...] - j * BN
        cols = lax.broadcasted_iota(jnp.int32, (B, BN), 1)
        hit = ((blk > tval) & (cols < N - (NBLK - 1) * BN)) | (
            (blk == tval) & (cols < tgt_adj)
        )
        acc_ref[...] += jnp.sum(hit.astype(jnp.int32), axis=1, keepdims=True)
        correct = (acc_ref[...] < TOPK).astype(jnp.float32)
        out_ref[...] = jnp.sum(correct, axis=(0, 1), keepdims=True) * (1.0 / B)


def kernel(preds, target):
    tvals = _gather_tvals(preds, target)
    out = pl.pallas_call(
        _count_body,
        grid=(NBLK,),
        in_specs=[
            pl.BlockSpec((B, BN), lambda j: (0, j)),
            pl.BlockSpec((B, 1), lambda j: (0, 0)),
            pl.BlockSpec((B, 1), lambda j: (0, 0)),
        ],
        out_specs=pl.BlockSpec((1, 1), lambda j: (0, 0)),
        out_shape=jax.ShapeDtypeStruct((1, 1), jnp.float32),
        scratch_shapes=[pltpu.VMEM((B, 1), jnp.int32)],
    )(preds, tvals.reshape(B, 1), target.reshape(B, 1).astype(jnp.int32))
    return out[0, 0]
